# baseline (speedup 1.0000x reference)
import jax
import jax.numpy as jnp
from jax import lax
from jax.experimental import pallas as pl

F32 = jnp.float32
NB = 1024
CDIM = 512
BATCH = 32
TLEN = 64
FRAME = 263
BETA = 1.0
DILS = (9, 3, 1)


def _riota(m):
    return lax.broadcasted_iota(jnp.int32, (m, 1), 0)


def _shift_mask(h, s, t):
    m = h.shape[0]
    if s == 0:
        return h
    if abs(s) >= t:
        return None
    sm = s % m
    a = jnp.concatenate([h[sm:], h[:sm]], axis=0)
    pos = lax.bitwise_and(_riota(m), t - 1)
    valid = jnp.logical_and(pos + s >= 0, pos + s < t)
    return jnp.where(valid, a, jnp.zeros_like(a))


def _mm(a, b):
    return lax.dot_general(a, b, (((1,), (0,)), ((), ())),
                           preferred_element_type=F32)


def _convk(h, w_ref, b_ref, t, dil):
    k = w_ref.shape[0]
    acc = None
    for j in range(k):
        s = (j - k // 2) * dil
        a = _shift_mask(h, s, t)
        if a is None:
            continue
        v = _mm(a, w_ref[j])
        acc = v if acc is None else acc + v
    if acc is None:
        acc = jnp.zeros((h.shape[0], w_ref.shape[2]), F32)
    return acc + b_ref[...]


def _resnet(h, rrefs, t):
    for (w1, b1, w2, b2), d in zip(rrefs, DILS):
        m = jnp.maximum(h, 0.0)
        m = _convk(m, w1, b1, t, d)
        m = jnp.maximum(m, 0.0)
        m = _convk(m, w2, b2, t, 1)
        h = h + m
    return h


def _group_res(refs):
    return [tuple(refs[4 * i:4 * i + 4]) for i in range(3)]


def _k_in(x_ref, w_ref, b_ref, o_ref):
    h = _convk(x_ref[...], w_ref, b_ref, TLEN, 1)
    o_ref[...] = jnp.maximum(h, 0.0)


def _k_down(t_out, has_out, *refs):
    if has_out:
        (he_ref, ho_ref, dw, db, *rest) = refs
        res_refs = rest[:12]
        ow, ob, o_ref = rest[12], rest[13], rest[14]
    else:
        (he_ref, ho_ref, dw, db, *rest) = refs
        res_refs = rest[:12]
        o_ref = rest[12]
    he = he_ref[...]
    ho = ho_ref[...]
    h = _mm(he, dw[1]) + _mm(ho, dw[2]) + db[...]
    a = _shift_mask(ho, -1, t_out)
    if a is not None:
        h = h + _mm(a, dw[0])
    a = _shift_mask(he, 1, t_out)
    if a is not None:
        h = h + _mm(a, dw[3])
    h = _resnet(h, _group_res(res_refs), t_out)
    if has_out:
        h = _convk(h, ow, ob, t_out, 1)
    o_ref[...] = h


def _k_quant(z_ref, cb_ref, cbt_ref, zq_ref, loss_ref, perp_ref):
    zf = z_ref[...]
    cbt = cbt_ref[...]
    n = zf.shape[0]
    z2 = jnp.sum(zf * zf, axis=1, keepdims=True)
    c2 = jnp.sum(cbt * cbt, axis=0, keepdims=True)
    d = z2 + c2 - 2.0 * _mm(zf, cbt)
    dmin = jnp.min(d, axis=1, keepdims=True)
    kio = lax.broadcasted_iota(jnp.int32, d.shape, 1)
    idx = jnp.min(jnp.where(d == dmin, kio, NB), axis=1, keepdims=True)
    oh = (kio == idx).astype(F32)
    zq = _mm(oh, cb_ref[...])
    zq_ref[...] = zq
    df = zq - zf
    sq = jnp.sum(df * df, axis=1, keepdims=True)
    tot = jnp.sum(sq, axis=0, keepdims=True)
    loss_ref[...] = ((1.0 + BETA) / (n * zf.shape[1])) * tot
    em = jnp.sum(oh, axis=0, keepdims=True) * (1.0 / n)
    ent = jnp.sum(em * jnp.log(em + 1e-10), axis=1, keepdims=True)
    perp_ref[...] = jnp.exp(-ent)


def _k_up(t_in, has_in, *refs):
    if has_in:
        (x_ref, iw, ib, *rest) = refs
    else:
        (x_ref, *rest) = refs
    res_refs = rest[:12]
    uw, ub, o_ref = rest[12], rest[13], rest[14]
    h = x_ref[...]
    if has_in:
        h = jnp.maximum(_convk(h, iw, ib, t_in, 1), 0.0)
    h = _resnet(h, _group_res(res_refs), t_in)
    w0, w1, w2 = uw[0], uw[1], uw[2]
    b = ub[...]
    ev = _mm(h, w1 + w2) + b
    a = _shift_mask(h, -1, t_in)
    if a is not None:
        ev = ev + _mm(a, w0)
    od = _mm(h, w0 + w1) + b
    a = _shift_mask(h, 1, t_in)
    if a is not None:
        od = od + _mm(a, w2)
    o_ref[:, 0, :] = ev
    o_ref[:, 1, :] = od


def _k_out(x_ref, w1, b1, w2, b2, o_ref):
    h = jnp.maximum(_convk(x_ref[...], w1, b1, TLEN, 1), 0.0)
    o_ref[...] = _convk(h, w2, b2, TLEN, 1)


def _pc(fn, out_shape):
    return pl.pallas_call(fn, out_shape=out_shape)


def _split(h):
    h3 = h.reshape(-1, 2, h.shape[1])
    return h3[:, 0, :], h3[:, 1, :]


def _tw(w):
    return jnp.transpose(w, (2, 1, 0))


def _rb(b):
    return b.reshape(1, -1)


def _res_args(res):
    out = []
    for p in res:
        out += [_tw(p['w1']), _rb(p['b1']), _tw(p['w2']), _rb(p['b2'])]
    return out


def kernel(x, params):
    enc = params['encoder']
    dec = params['decoder']
    cb = params['codebook']
    m0 = BATCH * TLEN

    xf = x.reshape(m0, FRAME).astype(F32)

    h = _pc(_k_in, jax.ShapeDtypeStruct((m0, CDIM), F32))(
        xf, _tw(enc['in_w']), _rb(enc['in_b']))

    t = TLEN
    for i, blk in enumerate(enc['down']):
        t //= 2
        m = BATCH * t
        he, ho = _split(h)
        args = [he, ho, _tw(blk['dw']), _rb(blk['db'])] + _res_args(blk['res'])
        has_out = i == 2
        if has_out:
            args += [_tw(enc['out_w']), _rb(enc['out_b'])]

        def body(*refs, _t=t, _ho=has_out):
            _k_down(_t, _ho, *refs)

        h = _pc(body, jax.ShapeDtypeStruct((m, CDIM), F32))(*args)

    zq, loss, perp = _pc(
        _k_quant,
        (jax.ShapeDtypeStruct((BATCH * t, CDIM), F32),
         jax.ShapeDtypeStruct((1, 1), F32),
         jax.ShapeDtypeStruct((1, 1), F32)))(h, cb, cb.T)

    h = zq
    for i, blk in enumerate(dec['up']):
        m = h.shape[0]
        has_in = i == 0
        args = [h]
        if has_in:
            args += [_tw(dec['in_w']), _rb(dec['in_b'])]
        args += _res_args(blk['res']) + [_tw(blk['uw']), _rb(blk['ub'])]

        def body(*refs, _t=t, _hi=has_in):
            _k_up(_t, _hi, *refs)

        h = _pc(body, jax.ShapeDtypeStruct((m, 2, CDIM), F32))(*args)
        h = h.reshape(2 * m, CDIM)
        t *= 2

    xo = _pc(_k_out, jax.ShapeDtypeStruct((m0, FRAME), F32))(
        h, _tw(dec['out1_w']), _rb(dec['out1_b']),
        _tw(dec['out2_w']), _rb(dec['out2_b']))

    return (xo.reshape(BATCH, TLEN, FRAME),
            loss.reshape(()), perp.reshape(()))


# speedup vs baseline: 1.0917x; 1.0917x over previous
import jax
import jax.numpy as jnp
from jax import lax
from jax.experimental import pallas as pl

F32 = jnp.float32
NB = 1024
CDIM = 512
BATCH = 32
TLEN = 64
FRAME = 263
BETA = 1.0
DILS = (9, 3, 1)


def _riota(m):
    return lax.broadcasted_iota(jnp.int32, (m, 1), 0)


def _shift_mask(h, s, t):
    m = h.shape[0]
    if s == 0:
        return h
    if abs(s) >= t:
        return None
    sm = s % m
    a = jnp.concatenate([h[sm:], h[:sm]], axis=0)
    pos = lax.bitwise_and(_riota(m), t - 1)
    valid = jnp.logical_and(pos + s >= 0, pos + s < t)
    return jnp.where(valid, a, jnp.zeros_like(a))


def _mm(a, b):
    return lax.dot_general(a, b, (((1,), (0,)), ((), ())),
                           preferred_element_type=F32)


def _mm_t(a, b):
    return lax.dot_general(a, b, (((1,), (1,)), ((), ())),
                           preferred_element_type=F32)


def _convk(h, w_ref, b_ref, t, dil):
    k = w_ref.shape[0]
    acc = None
    for j in range(k):
        s = (j - k // 2) * dil
        a = _shift_mask(h, s, t)
        if a is None:
            continue
        v = _mm(a, w_ref[j])
        acc = v if acc is None else acc + v
    if acc is None:
        acc = jnp.zeros((h.shape[0], w_ref.shape[2]), F32)
    return acc + b_ref[...]


def _resnet(h, rrefs, t):
    for (w1, b1, w2, b2), d in zip(rrefs, DILS):
        m = jnp.maximum(h, 0.0)
        m = _convk(m, w1, b1, t, d)
        m = jnp.maximum(m, 0.0)
        m = _mm_t(m, w2[...]) + b2[...]
        h = h + m
    return h


def _group_res(refs):
    return [tuple(refs[4 * i:4 * i + 4]) for i in range(3)]


def _k_in(x_ref, w_ref, b_ref, o_ref):
    h = _convk(x_ref[...], w_ref, b_ref, TLEN, 1)
    o_ref[...] = jnp.maximum(h, 0.0)


def _k_down(t_out, has_out, *refs):
    if has_out:
        (he_ref, ho_ref, dw, db, *rest) = refs
        res_refs = rest[:12]
        ow, ob, o_ref = rest[12], rest[13], rest[14]
    else:
        (he_ref, ho_ref, dw, db, *rest) = refs
        res_refs = rest[:12]
        o_ref = rest[12]
    he = he_ref[...]
    ho = ho_ref[...]
    h = _mm(he, dw[1]) + _mm(ho, dw[2]) + db[...]
    a = _shift_mask(ho, -1, t_out)
    if a is not None:
        h = h + _mm(a, dw[0])
    a = _shift_mask(he, 1, t_out)
    if a is not None:
        h = h + _mm(a, dw[3])
    h = _resnet(h, _group_res(res_refs), t_out)
    if has_out:
        h = _convk(h, ow, ob, t_out, 1)
    o_ref[...] = h


def _k_quant(z_ref, cb_ref, cbt_ref, zq_ref, loss_ref, perp_ref):
    zf = z_ref[...]
    cbt = cbt_ref[...]
    n = zf.shape[0]
    z2 = jnp.sum(zf * zf, axis=1, keepdims=True)
    c2 = jnp.sum(cbt * cbt, axis=0, keepdims=True)
    d = z2 + c2 - 2.0 * _mm(zf, cbt)
    dmin = jnp.min(d, axis=1, keepdims=True)
    kio = lax.broadcasted_iota(jnp.int32, d.shape, 1)
    idx = jnp.min(jnp.where(d == dmin, kio, NB), axis=1, keepdims=True)
    oh = (kio == idx).astype(F32)
    zq = _mm(oh, cb_ref[...])
    zq_ref[...] = zq
    df = zq - zf
    sq = jnp.sum(df * df, axis=1, keepdims=True)
    tot = jnp.sum(sq, axis=0, keepdims=True)
    loss_ref[...] = ((1.0 + BETA) / (n * zf.shape[1])) * tot
    em = jnp.sum(oh, axis=0, keepdims=True) * (1.0 / n)
    ent = jnp.sum(em * jnp.log(em + 1e-10), axis=1, keepdims=True)
    perp_ref[...] = jnp.exp(-ent)


def _k_up(t_in, has_in, *refs):
    if has_in:
        (x_ref, iw, ib, *rest) = refs
    else:
        (x_ref, *rest) = refs
    res_refs = rest[:12]
    uw, ub, o_ref = rest[12], rest[13], rest[14]
    h = x_ref[...]
    if has_in:
        h = jnp.maximum(_convk(h, iw, ib, t_in, 1), 0.0)
    h = _resnet(h, _group_res(res_refs), t_in)
    w0, w1, w2 = uw[0], uw[1], uw[2]
    b = ub[...]
    ev = _mm(h, w1 + w2) + b
    a = _shift_mask(h, -1, t_in)
    if a is not None:
        ev = ev + _mm(a, w0)
    od = _mm(h, w0 + w1) + b
    a = _shift_mask(h, 1, t_in)
    if a is not None:
        od = od + _mm(a, w2)
    o_ref[:, 0, :] = ev
    o_ref[:, 1, :] = od


def _k_out(x_ref, w1, b1, w2, b2, o_ref):
    h = jnp.maximum(_convk(x_ref[...], w1, b1, TLEN, 1), 0.0)
    o_ref[...] = _convk(h, w2, b2, TLEN, 1)


def _pc(fn, out_shape):
    return pl.pallas_call(fn, out_shape=out_shape)


def _split(h):
    h3 = h.reshape(-1, 2, h.shape[1])
    return h3[:, 0, :], h3[:, 1, :]


def _tw(w):
    return jnp.transpose(w, (2, 1, 0))


def _rb(b):
    return b.reshape(1, -1)


def _res_args(res):
    out = []
    for p in res:
        w2 = p['w2']
        out += [_tw(p['w1']), _rb(p['b1']),
                w2.reshape(w2.shape[0], w2.shape[1]), _rb(p['b2'])]
    return out


def kernel(x, params):
    enc = params['encoder']
    dec = params['decoder']
    cb = params['codebook']
    m0 = BATCH * TLEN

    xf = x.reshape(m0, FRAME).astype(F32)

    h = _pc(_k_in, jax.ShapeDtypeStruct((m0, CDIM), F32))(
        xf, _tw(enc['in_w']), _rb(enc['in_b']))

    t = TLEN
    for i, blk in enumerate(enc['down']):
        t //= 2
        m = BATCH * t
        he, ho = _split(h)
        args = [he, ho, _tw(blk['dw']), _rb(blk['db'])] + _res_args(blk['res'])
        has_out = i == 2
        if has_out:
            args += [_tw(enc['out_w']), _rb(enc['out_b'])]

        def body(*refs, _t=t, _ho=has_out):
            _k_down(_t, _ho, *refs)

        h = _pc(body, jax.ShapeDtypeStruct((m, CDIM), F32))(*args)

    zq, loss, perp = _pc(
        _k_quant,
        (jax.ShapeDtypeStruct((BATCH * t, CDIM), F32),
         jax.ShapeDtypeStruct((1, 1), F32),
         jax.ShapeDtypeStruct((1, 1), F32)))(h, cb, cb.T)

    h = zq
    for i, blk in enumerate(dec['up']):
        m = h.shape[0]
        has_in = i == 0
        args = [h]
        if has_in:
            args += [_tw(dec['in_w']), _rb(dec['in_b'])]
        args += _res_args(blk['res']) + [_tw(blk['uw']), _rb(blk['ub'])]

        def body(*refs, _t=t, _hi=has_in):
            _k_up(_t, _hi, *refs)

        h = _pc(body, jax.ShapeDtypeStruct((m, 2, CDIM), F32))(*args)
        h = h.reshape(2 * m, CDIM)
        t *= 2

    xo = _pc(_k_out, jax.ShapeDtypeStruct((m0, FRAME), F32))(
        h, _tw(dec['out1_w']), _rb(dec['out1_b']),
        _tw(dec['out2_w']), _rb(dec['out2_b']))

    return (xo.reshape(BATCH, TLEN, FRAME),
            loss.reshape(()), perp.reshape(()))
